# baseline (speedup 1.0000x reference)
import jax
import jax.numpy as jnp
from jax import lax
from jax.experimental import pallas as pl
from jax.experimental.pallas import tpu as pltpu
from jax.experimental.pallas import tpu_sc as plsc

VOCAB_N = 1000000
D_MODEL = 64
SCALE = 8.0

A_VBLK = 2048

W = 128
K = 2
SUPER = W * K
NUM_WORKERS = 32

C_BBLK = 256


def _format_table(table_t):
    n_v = table_t.shape[1]

    def body(in_ref, out_ref):
        y = in_ref[...].T * SCALE
        out_ref[...] = jnp.concatenate([y, jnp.zeros_like(y)], axis=1)

    grid = (n_v + A_VBLK - 1) // A_VBLK
    return pl.pallas_call(
        body,
        grid=(grid,),
        in_specs=[pl.BlockSpec((D_MODEL, A_VBLK), lambda i: (0, i))],
        out_specs=pl.BlockSpec((A_VBLK, 128), lambda i: (i, 0)),
        out_shape=jax.ShapeDtypeStruct((n_v, 128), jnp.float32),
    )(table_t)


def _sc_gather(table_p, idx):
    n = idx.shape[0]
    per_w = n // NUM_WORKERS
    nsup = per_w // SUPER

    mesh = plsc.VectorSubcoreMesh(core_axis_name="c", subcore_axis_name="s")

    @pl.kernel(
        out_type=jax.ShapeDtypeStruct((n, D_MODEL), jnp.float32),
        mesh=mesh,
        compiler_params=pltpu.CompilerParams(use_tc_tiling_on_sc=False),
        scratch_types=[
            pltpu.VMEM((2, SUPER), jnp.int32),
            pltpu.VMEM((2, SUPER, 128), jnp.float32),
            pltpu.SemaphoreType.DMA,
            pltpu.SemaphoreType.DMA,
            pltpu.SemaphoreType.DMA,
            pltpu.SemaphoreType.DMA,
        ],
    )
    def emb_kernel(table_hbm, idx_hbm, out_hbm, idx_v, rows_v, g0, g1, w0, w1):
        wid = lax.axis_index("s") * 2 + lax.axis_index("c")
        wbase = wid * per_w
        gsems = (g0, g1)
        wsems = (w0, w1)

        def load_idx(c, b):
            pltpu.sync_copy(
                idx_hbm.at[pl.ds(wbase + c * SUPER, SUPER)], idx_v.at[b]
            )

        def fire_gathers(b):
            for w in range(K):
                pltpu.async_copy(
                    table_hbm.at[idx_v.at[b, pl.ds(w * W, W)]],
                    rows_v.at[b, pl.ds(w * W, W)],
                    gsems[b],
                )

        def drain_gathers(b):
            pltpu.make_async_copy(
                table_hbm.at[pl.ds(0, SUPER)], rows_v.at[b], gsems[b]
            ).wait()

        def fire_writeback(c, b):
            pltpu.async_copy(
                rows_v.at[b, :, pl.ds(0, D_MODEL)],
                out_hbm.at[pl.ds(wbase + c * SUPER, SUPER)],
                wsems[b],
            )

        def drain_writeback(b):
            pltpu.make_async_copy(
                out_hbm.at[pl.ds(0, SUPER)],
                rows_v.at[b, :, pl.ds(0, D_MODEL)],
                wsems[b],
            ).wait()

        load_idx(0, 0)
        fire_gathers(0)

        @pl.loop(0, nsup // 2)
        def _(i):
            for b in (0, 1):
                c = i * 2 + b
                nb = 1 - b

                @pl.when(c + 1 < nsup)
                def _():
                    load_idx(c + 1, nb)

                    @pl.when(c >= 1)
                    def _():
                        drain_writeback(nb)

                    fire_gathers(nb)

                drain_gathers(b)
                fire_writeback(c, b)

        drain_writeback(0)
        drain_writeback(1)

    return emb_kernel(table_p, idx)


def _finalize(g2, batch_n):
    rows_per_b = 50 * D_MODEL // 128

    def body(in_ref, out_ref):
        x = in_ref[...]
        z = x.reshape(C_BBLK, 50 * D_MODEL)
        out_ref[...] = z.T.reshape(50, D_MODEL, C_BBLK)

    grid = batch_n // C_BBLK
    return pl.pallas_call(
        body,
        grid=(grid,),
        in_specs=[pl.BlockSpec((C_BBLK * rows_per_b, 128), lambda i: (i, 0))],
        out_specs=pl.BlockSpec((50, D_MODEL, C_BBLK), lambda i: (0, 0, i)),
        out_shape=jax.ShapeDtypeStruct((50, D_MODEL, batch_n), jnp.float32),
    )(g2)


def kernel(inputs, table):
    B, S = inputs.shape
    n = B * S
    idx = inputs.reshape(n).astype(jnp.int32)

    table_p = _format_table(table.T)

    gathered = _sc_gather(table_p, idx)

    g2 = gathered.reshape(n // 2, 128)
    out_t = _finalize(g2, B)
    return out_t.transpose(2, 0, 1)


# speedup vs baseline: 1.1680x; 1.1680x over previous
import jax
import jax.numpy as jnp
from jax import lax
from jax.experimental import pallas as pl
from jax.experimental.pallas import tpu as pltpu
from jax.experimental.pallas import tpu_sc as plsc

VOCAB_N = 1000000
HALF_V = VOCAB_N // 2
D_MODEL = 64
SCALE = 8.0

A_VBLK = 1024
A_GRID = (VOCAB_N + 2 * A_VBLK - 1) // (2 * A_VBLK)
N_PAIR = A_GRID * A_VBLK

W = 128
K = 4
SUPER = W * K
NUM_WORKERS = 32

C_BBLK = 256


def _format_table(table_t):
    def body(in_ref, out_ref):
        x = in_ref[...]
        lo = x[:, :A_VBLK].T * SCALE
        hi = x[:, A_VBLK:].T * SCALE
        out_ref[...] = jnp.concatenate([lo, hi], axis=1)

    return pl.pallas_call(
        body,
        grid=(A_GRID,),
        in_specs=[pl.BlockSpec((D_MODEL, 2 * A_VBLK), lambda i: (0, i))],
        out_specs=pl.BlockSpec((A_VBLK, 128), lambda i: (i, 0)),
        out_shape=jax.ShapeDtypeStruct((N_PAIR, 128), jnp.float32),
    )(table_t)


def _sc_gather(table_l, idx):
    n = idx.shape[0]
    per_w = n // NUM_WORKERS
    nsup = per_w // SUPER

    mesh = plsc.VectorSubcoreMesh(core_axis_name="c", subcore_axis_name="s")

    @pl.kernel(
        out_type=jax.ShapeDtypeStruct((n, D_MODEL), jnp.float32),
        mesh=mesh,
        compiler_params=pltpu.CompilerParams(use_tc_tiling_on_sc=False),
        scratch_types=[
            pltpu.VMEM((2, SUPER), jnp.int32),
            pltpu.VMEM((2, SUPER, D_MODEL), jnp.float32),
            pltpu.SemaphoreType.DMA,
            pltpu.SemaphoreType.DMA,
            pltpu.SemaphoreType.DMA,
            pltpu.SemaphoreType.DMA,
        ],
    )
    def emb_kernel(table_hbm, idx_hbm, out_hbm, idx_v, rows_v, g0, g1, w0, w1):
        wid = lax.axis_index("s") * 2 + lax.axis_index("c")
        wbase = wid * per_w
        gsems = (g0, g1)
        wsems = (w0, w1)

        def load_idx(c, b):
            pltpu.sync_copy(
                idx_hbm.at[pl.ds(wbase + c * SUPER, SUPER)], idx_v.at[b]
            )

        def fire_gathers(b):
            for w in range(K):
                pltpu.async_copy(
                    table_hbm.at[idx_v.at[b, pl.ds(w * W, W)]],
                    rows_v.at[b, pl.ds(w * W, W)],
                    gsems[b],
                )

        def drain_gathers(b):
            pltpu.make_async_copy(
                table_hbm.at[pl.ds(0, SUPER)], rows_v.at[b], gsems[b]
            ).wait()

        def fire_writeback(c, b):
            pltpu.async_copy(
                rows_v.at[b],
                out_hbm.at[pl.ds(wbase + c * SUPER, SUPER)],
                wsems[b],
            )

        def drain_writeback(b):
            pltpu.make_async_copy(
                out_hbm.at[pl.ds(0, SUPER)], rows_v.at[b], wsems[b]
            ).wait()

        load_idx(0, 0)
        fire_gathers(0)

        @pl.loop(0, nsup // 2)
        def _(i):
            for b in (0, 1):
                c = i * 2 + b
                nb = 1 - b

                @pl.when(c + 1 < nsup)
                def _():
                    load_idx(c + 1, nb)

                    @pl.when(c >= 1)
                    def _():
                        drain_writeback(nb)

                    fire_gathers(nb)

                drain_gathers(b)
                fire_writeback(c, b)

        drain_writeback(0)
        drain_writeback(1)

    return emb_kernel(table_l, idx)


def _finalize(g2, batch_n):
    rows_per_b = 50 * D_MODEL // 128

    def body(in_ref, out_ref):
        x = in_ref[...]
        z = x.reshape(C_BBLK, 50 * D_MODEL)
        out_ref[...] = z.T.reshape(50, D_MODEL, C_BBLK)

    grid = batch_n // C_BBLK
    return pl.pallas_call(
        body,
        grid=(grid,),
        in_specs=[pl.BlockSpec((C_BBLK * rows_per_b, 128), lambda i: (i, 0))],
        out_specs=pl.BlockSpec((50, D_MODEL, C_BBLK), lambda i: (0, 0, i)),
        out_shape=jax.ShapeDtypeStruct((50, D_MODEL, batch_n), jnp.float32),
    )(g2)


def kernel(inputs, table):
    B, S = inputs.shape
    n = B * S
    idx = inputs.reshape(n).astype(jnp.int32)
    idx = 2 * ((idx >> 11) * A_VBLK + (idx & (A_VBLK - 1))) + (
        (idx >> 10) & 1
    )

    table_p = _format_table(table.T)
    table_l = table_p.reshape(2 * N_PAIR, D_MODEL)

    gathered = _sc_gather(table_l, idx)

    g2 = gathered.reshape(n // 2, 128)
    out_t = _finalize(g2, B)
    return out_t.transpose(2, 0, 1)
